# baseline (speedup 1.0000x reference)
import functools

import jax
import jax.numpy as jnp
import numpy as np
from jax import lax
from jax.experimental import pallas as pl
from jax.experimental.pallas import tpu as pltpu
from jax.experimental.pallas import tpu_sc as plsc

N = 10000
E = 320000
D = 128
ED = 16
NC, NS = 2, 16
NW = NC * NS
EPT = E // NW
BK = 40
NBLK = EPT // BK
DP = D + 16
NA = 10240
RPT = NA // NS

_SILU_C = (0.0005758678889833391, 0.2481342852115631, -0.019295798614621162,
           0.001511988928541541, -8.809041901258752e-05, 3.3523247111588717e-06,
           -7.264971912945839e-08, 6.729672374916618e-10)


_PERM = np.concatenate([
    np.concatenate([np.arange(16 * g, 16 * g + 16),
                    np.arange(64 + 16 * g, 64 + 16 * g + 16)])
    for g in range(D // 32)
]).astype(np.int32)


def _silu_poly(x):
    c0, c1, c2, c3, c4, c5, c6, c7 = _SILU_C
    u = jnp.minimum(x * x, 25.0)
    u2 = u * u
    u4 = u2 * u2
    lo = (c0 + c1 * u) + u2 * (c2 + c3 * u)
    hi = (c4 + c5 * u) + u2 * (c6 + c7 * u)
    p = lo + u4 * hi
    return p + 0.5 * x + 0.5 * jnp.maximum(jnp.abs(x) - 5.0, 0.0)
NB = 2000
EB = 8000

_HI = lax.Precision.HIGHEST


def _dot(a, b):
    return jnp.dot(a, b, preferred_element_type=jnp.float32, precision=_HI)


def _dotf(a, b):
    return jnp.dot(a, b, preferred_element_type=jnp.float32)


def _pack_pair(lo_f32, hi_f32):
    lo = lax.bitcast_convert_type(lo_f32.astype(jnp.bfloat16), jnp.uint16)
    hi = lax.bitcast_convert_type(hi_f32.astype(jnp.bfloat16), jnp.uint16)
    word = lo.astype(jnp.uint32) | (hi.astype(jnp.uint32) << 16)
    return lax.bitcast_convert_type(word, jnp.float32)


def _c_body(ea_ref, w_ref, b_ref, c0_ref, c1_ref):
    ea = ea_ref[...]
    for i, cref in ((0, c0_ref), (1, c1_ref)):
        lo = _dotf(ea, w_ref[i, :, :64]) + b_ref[i, :64]
        hi = _dotf(ea, w_ref[i, :, 64:]) + b_ref[i, 64:]
        cref[...] = _pack_pair(lo, hi)


def _tc_edge_bias(edge_attr, W1c, eb1):
    return pl.pallas_call(
        _c_body,
        grid=(E // EB,),
        in_specs=[
            pl.BlockSpec((EB, ED), lambda i: (i, 0)),
            pl.BlockSpec((2, ED, D), lambda i: (0, 0, 0)),
            pl.BlockSpec((2, D), lambda i: (0, 0)),
        ],
        out_specs=[
            pl.BlockSpec((EB, D // 2), lambda i: (i, 0)),
            pl.BlockSpec((EB, D // 2), lambda i: (i, 0)),
        ],
        out_shape=[
            jax.ShapeDtypeStruct((E, D // 2), jnp.float32),
            jax.ShapeDtypeStruct((E, D // 2), jnp.float32),
        ],
    )(edge_attr, W1c, eb1)


def _pre_body(h_ref, g_ref, b_ref, wa_ref, wb_ref, hn_ref, a_ref, bb_ref):
    x = h_ref[...]
    m = jnp.mean(x, axis=-1, keepdims=True)
    v = jnp.mean(jnp.square(x - m), axis=-1, keepdims=True)
    hn = (x - m) / jnp.sqrt(v + 1e-5) * g_ref[...] + b_ref[...]
    hn_ref[...] = hn
    wa = wa_ref[...]
    wb = wb_ref[...]
    a_ref[...] = _pack_pair(_dotf(hn, wa[:, :64]), _dotf(hn, wa[:, 64:]))
    bb_ref[...] = _pack_pair(_dotf(hn, wb[:, :64]), _dotf(hn, wb[:, 64:]))


def _tc_pre(h, g1, b1, W1a, W1b):
    return pl.pallas_call(
        _pre_body,
        grid=(N // NB,),
        in_specs=[
            pl.BlockSpec((NB, D), lambda i: (i, 0)),
            pl.BlockSpec((1, D), lambda i: (0, 0)),
            pl.BlockSpec((1, D), lambda i: (0, 0)),
            pl.BlockSpec((D, D), lambda i: (0, 0)),
            pl.BlockSpec((D, D), lambda i: (0, 0)),
        ],
        out_specs=[
            pl.BlockSpec((NB, D), lambda i: (i, 0)),
            pl.BlockSpec((NB, D // 2), lambda i: (i, 0)),
            pl.BlockSpec((NB, D // 2), lambda i: (i, 0)),
        ],
        out_shape=[
            jax.ShapeDtypeStruct((N, D), jnp.float32),
            jax.ShapeDtypeStruct((N, D // 2), jnp.float32),
            jax.ShapeDtypeStruct((N, D // 2), jnp.float32),
        ],
    )(h, g1.reshape(1, D), b1.reshape(1, D), W1a, W1b)


def _sc_body(a_hbm, b_hbm, c_hbm, src_hbm, dst_hbm, out_hbm,
             sidx0, sidx1, sidx2, sidx3, didx0, didx1, didx2, didx3,
             bufa0, bufa1, bufb0, bufb1, bufc0, bufc1, buft0, buft1, acc,
             ssi0, ssi1, ssi2, ssi3, sdi0, sdi1, sdi2, sdi3,
             sa0, sa1, sb0, sb1, sc0, sc1, sst0, sst1):
    sidx = (sidx0, sidx1, sidx2, sidx3)
    didx = (didx0, didx1, didx2, didx3)
    bufa = (bufa0, bufa1)
    bufb = (bufb0, bufb1)
    bufc = (bufc0, bufc1)
    buft = (buft0, buft1)
    sem_si = (ssi0, ssi1, ssi2, ssi3)
    sem_di = (sdi0, sdi1, sdi2, sdi3)
    sem_a = (sa0, sa1)
    sem_b = (sb0, sb1)
    sem_c = (sc0, sc1)
    sem_s = (sst0, sst1)

    cid = lax.axis_index("c")
    sid = lax.axis_index("s")
    zeros16 = jnp.zeros((16,), jnp.float32)

    def zrow(r, carry):
        for j in range(DP // 16):
            buft0[r, pl.ds(j * 16, 16)] = zeros16
        return carry

    lax.fori_loop(0, BK, zrow, 0)

    row0 = sid * RPT
    for q in range(RPT // BK):
        pltpu.sync_copy(buft0, acc.at[pl.ds(row0 + q * BK, BK)])

    onehot = jnp.where(lax.iota(jnp.int32, 16) == 0,
                       jnp.float32(1.0), jnp.float32(0.0))

    def trow(r, carry):
        buft0[r, pl.ds(D, 16)] = onehot
        buft1[r, pl.ds(D, 16)] = onehot
        return carry

    lax.fori_loop(0, BK, trow, 0)

    plsc.subcore_barrier()

    base = (cid * NS + sid) * EPT

    def issue_idx(jb, q):
        off = base + jb * BK
        pltpu.async_copy(src_hbm.at[pl.ds(off, BK)], sidx[q], sem_si[q])
        pltpu.async_copy(dst_hbm.at[pl.ds(off, BK)], didx[q], sem_di[q])

    def wait_idx(q):
        pltpu.make_async_copy(src_hbm.at[pl.ds(0, BK)], sidx[q],
                              sem_si[q]).wait()
        pltpu.make_async_copy(dst_hbm.at[pl.ds(0, BK)], didx[q],
                              sem_di[q]).wait()

    def issue_gathers(jb, q, p):
        off = base + jb * BK
        pltpu.async_copy(a_hbm.at[sidx[q]], bufa[p], sem_a[p])
        pltpu.async_copy(b_hbm.at[didx[q]], bufb[p], sem_b[p])
        pltpu.async_copy(c_hbm.at[pl.ds(off, BK)], bufc[p], sem_c[p])

    def wait_gathers(p):
        pltpu.make_async_copy(a_hbm.at[sidx[0]], bufa[p], sem_a[p]).wait()
        pltpu.make_async_copy(b_hbm.at[didx[0]], bufb[p], sem_b[p]).wait()
        pltpu.make_async_copy(c_hbm.at[pl.ds(0, BK)], bufc[p], sem_c[p]).wait()

    def issue_scatter(q, p):
        pltpu.async_copy(buft[p], acc.at[didx[q]], sem_s[p], add=True)

    def wait_scatter(p):
        pltpu.make_async_copy(buft[p], acc.at[didx[0]], sem_s[p]).wait()

    def compute(p):
        @plsc.parallel_loop(0, BK, step=1, unroll=8)
        def erow(e):
            fmt = plsc.PackFormat.INTERLEAVED
            for g in range(D // 32):
                sl = pl.ds(16 * g, 16)
                a0, a1 = plsc.unpack(
                    plsc.bitcast(bufa[p][e, sl], jnp.bfloat16), format=fmt)
                b0, b1 = plsc.unpack(
                    plsc.bitcast(bufb[p][e, sl], jnp.bfloat16), format=fmt)
                c0, c1 = plsc.unpack(
                    plsc.bitcast(bufc[p][e, sl], jnp.bfloat16), format=fmt)
                buft[p][e, pl.ds(32 * g, 16)] = _silu_poly(a0 + b0 + c0)
                buft[p][e, pl.ds(32 * g + 16, 16)] = _silu_poly(a1 + b1 + c1)

    def step(jb, q, p, guard_jo=None, issue_next=True, issue_idx2=True):
        qn = (q + 1) % 4
        pn = 1 - p
        if issue_next:
            wait_idx(qn)
        wait_gathers(p)
        if issue_next:
            issue_gathers(jb + 1, qn, pn)
        if guard_jo is None:
            wait_scatter(p)
        else:
            @pl.when(guard_jo >= 1)
            def _():
                wait_scatter(p)
        compute(p)
        issue_scatter(q, p)
        if issue_idx2:
            issue_idx(jb + 2, (q + 2) % 4)

    issue_idx(0, 0)
    wait_idx(0)
    issue_gathers(0, 0, 0)
    issue_idx(1, 1)

    def quad(jo, carry):
        for q in range(4):
            step(4 * jo + q, q, q % 2, guard_jo=jo if q < 2 else None)
        return carry

    lax.fori_loop(0, NBLK // 4, quad, 0)

    step(NBLK - 2, 0, 0, issue_idx2=False)
    step(NBLK - 1, 1, 1, issue_next=False, issue_idx2=False)
    wait_scatter(0)
    wait_scatter(1)

    plsc.subcore_barrier()
    pltpu.sync_copy(acc.at[pl.ds(row0, RPT)],
                    out_hbm.at[cid, pl.ds(row0, RPT)])


def _sc_edge(A, B, C, src, dst):
    mesh = plsc.VectorSubcoreMesh(core_axis_name="c", subcore_axis_name="s",
                                  num_cores=NC, num_subcores=NS)
    f = functools.partial(
        pl.kernel,
        out_type=jax.ShapeDtypeStruct((NC, NA, DP), jnp.float32),
        mesh=mesh,
        compiler_params=pltpu.CompilerParams(use_tc_tiling_on_sc=False,
                                             needs_layout_passes=False),
        scratch_types=[
            pltpu.VMEM((BK,), jnp.int32),
            pltpu.VMEM((BK,), jnp.int32),
            pltpu.VMEM((BK,), jnp.int32),
            pltpu.VMEM((BK,), jnp.int32),
            pltpu.VMEM((BK,), jnp.int32),
            pltpu.VMEM((BK,), jnp.int32),
            pltpu.VMEM((BK,), jnp.int32),
            pltpu.VMEM((BK,), jnp.int32),
            pltpu.VMEM((BK, D // 2), jnp.float32),
            pltpu.VMEM((BK, D // 2), jnp.float32),
            pltpu.VMEM((BK, D // 2), jnp.float32),
            pltpu.VMEM((BK, D // 2), jnp.float32),
            pltpu.VMEM((BK, D // 2), jnp.float32),
            pltpu.VMEM((BK, D // 2), jnp.float32),
            pltpu.VMEM((BK, DP), jnp.float32),
            pltpu.VMEM((BK, DP), jnp.float32),
            pltpu.VMEM_SHARED((NA, DP), jnp.float32),
        ] + [pltpu.SemaphoreType.DMA] * 16,
    )(_sc_body)
    return f(A, B, C, src, dst)


def _post_body(h_ref, hn_ref, p_ref, w2_ref, b2_ref, g2_ref, bb2_ref,
               nw1_ref, nb1_ref, nw2_ref, nb2_ref, o_ref):
    p = p_ref[0] + p_ref[1]
    t = p[:, :D]
    deg = p[:, D:D + 1]
    scale = 1.0 / jnp.maximum(deg, 1.0)
    agg = _dotf(t * scale, w2_ref[...]) + (deg * scale) * b2_ref[...]
    m = jnp.mean(agg, axis=-1, keepdims=True)
    v = jnp.mean(jnp.square(agg - m), axis=-1, keepdims=True)
    agg = (agg - m) / jnp.sqrt(v + 1e-5) * g2_ref[...] + bb2_ref[...]
    nw1 = nw1_ref[...]
    pre = _dotf(hn_ref[...], nw1[:D]) + _dotf(agg, nw1[D:]) + nb1_ref[...]
    act = pre * jax.nn.sigmoid(pre)
    o_ref[...] = h_ref[...] + _dotf(act, nw2_ref[...]) + nb2_ref[...]


def _tc_post(h, hn, P, eW2i, eb2i, g2, b2, nW1i, nb1i, nW2i, nb2i):
    return pl.pallas_call(
        _post_body,
        grid=(N // NB,),
        in_specs=[
            pl.BlockSpec((NB, D), lambda i: (i, 0)),
            pl.BlockSpec((NB, D), lambda i: (i, 0)),
            pl.BlockSpec((NC, NB, DP), lambda i: (0, i, 0)),
            pl.BlockSpec((D, D), lambda i: (0, 0)),
            pl.BlockSpec((1, D), lambda i: (0, 0)),
            pl.BlockSpec((1, D), lambda i: (0, 0)),
            pl.BlockSpec((1, D), lambda i: (0, 0)),
            pl.BlockSpec((2 * D, 2 * D), lambda i: (0, 0)),
            pl.BlockSpec((1, 2 * D), lambda i: (0, 0)),
            pl.BlockSpec((2 * D, D), lambda i: (0, 0)),
            pl.BlockSpec((1, D), lambda i: (0, 0)),
        ],
        out_specs=pl.BlockSpec((NB, D), lambda i: (i, 0)),
        out_shape=jax.ShapeDtypeStruct((N, D), jnp.float32),
    )(h, hn, P, eW2i, eb2i.reshape(1, D), g2.reshape(1, D), b2.reshape(1, D),
      nW1i, nb1i.reshape(1, 2 * D), nW2i, nb2i.reshape(1, D))


def kernel(node_state, edge_index, edge_attr, ln1_g, ln1_b, ln2_g, ln2_b,
           eW1, eb1, eW2, eb2, nW1, nb1, nW2, nb2):
    src = edge_index[0]
    dst = edge_index[1]
    W1c = eW1[:, 2 * D:, :]
    C01 = _tc_edge_bias(edge_attr, W1c, eb1)
    h = node_state
    for i in range(2):
        hn, A, B = _tc_pre(h, ln1_g[i], ln1_b[i], eW1[i, :D], eW1[i, D:2 * D])
        P = _sc_edge(A, B, C01[i], src, dst)
        h = _tc_post(h, hn, P, eW2[i][_PERM], eb2[i], ln2_g[i], ln2_b[i],
                     nW1[i], nb1[i], nW2[i], nb2[i])
    return h


# speedup vs baseline: 1.0659x; 1.0659x over previous
import functools

import jax
import jax.numpy as jnp
import numpy as np
from jax import lax
from jax.experimental import pallas as pl
from jax.experimental.pallas import tpu as pltpu
from jax.experimental.pallas import tpu_sc as plsc

N = 10000
E = 320000
D = 128
ED = 16
NC, NS = 2, 16
NW = NC * NS
EPT = E // NW
BK = 40
NBLK = EPT // BK
DP = D + 16
NA = 10240
RPT = NA // NS

_SILU_C = (0.0005758678889833391, 0.2481342852115631, -0.019295798614621162,
           0.001511988928541541, -8.809041901258752e-05, 3.3523247111588717e-06,
           -7.264971912945839e-08, 6.729672374916618e-10)


_PERM = np.concatenate([
    np.concatenate([np.arange(16 * g, 16 * g + 16),
                    np.arange(64 + 16 * g, 64 + 16 * g + 16)])
    for g in range(D // 32)
]).astype(np.int32)


def _silu_poly(x):
    c0, c1, c2, c3, c4, c5, c6, c7 = _SILU_C
    u = jnp.minimum(x * x, 25.0)
    u2 = u * u
    u4 = u2 * u2
    lo = (c0 + c1 * u) + u2 * (c2 + c3 * u)
    hi = (c4 + c5 * u) + u2 * (c6 + c7 * u)
    p = lo + u4 * hi
    return p + 0.5 * x + 0.5 * jnp.maximum(jnp.abs(x) - 5.0, 0.0)
NB = 2000
EB = 8000

_HI = lax.Precision.HIGHEST


def _dot(a, b):
    return jnp.dot(a, b, preferred_element_type=jnp.float32, precision=_HI)


def _dotf(a, b):
    return jnp.dot(a, b, preferred_element_type=jnp.float32)


def _pack_pair(lo_f32, hi_f32):
    lo = lax.bitcast_convert_type(lo_f32.astype(jnp.bfloat16), jnp.uint16)
    hi = lax.bitcast_convert_type(hi_f32.astype(jnp.bfloat16), jnp.uint16)
    word = lo.astype(jnp.uint32) | (hi.astype(jnp.uint32) << 16)
    return lax.bitcast_convert_type(word, jnp.float32)


def _c_body(ea_ref, w_ref, b_ref, c0_ref, c1_ref):
    ea = ea_ref[...]
    for i, cref in ((0, c0_ref), (1, c1_ref)):
        lo = _dotf(ea, w_ref[i, :, :64]) + b_ref[i, :64]
        hi = _dotf(ea, w_ref[i, :, 64:]) + b_ref[i, 64:]
        cref[...] = _pack_pair(lo, hi)


def _tc_edge_bias(edge_attr, W1c, eb1):
    return pl.pallas_call(
        _c_body,
        grid=(E // EB,),
        in_specs=[
            pl.BlockSpec((EB, ED), lambda i: (i, 0)),
            pl.BlockSpec((2, ED, D), lambda i: (0, 0, 0)),
            pl.BlockSpec((2, D), lambda i: (0, 0)),
        ],
        out_specs=[
            pl.BlockSpec((EB, D // 2), lambda i: (i, 0)),
            pl.BlockSpec((EB, D // 2), lambda i: (i, 0)),
        ],
        out_shape=[
            jax.ShapeDtypeStruct((E, D // 2), jnp.float32),
            jax.ShapeDtypeStruct((E, D // 2), jnp.float32),
        ],
    )(edge_attr, W1c, eb1)


def _pre_body(h_ref, g_ref, b_ref, wa_ref, wb_ref, hn_ref, a_ref, bb_ref):
    x = h_ref[...]
    m = jnp.mean(x, axis=-1, keepdims=True)
    v = jnp.mean(jnp.square(x - m), axis=-1, keepdims=True)
    hn = (x - m) / jnp.sqrt(v + 1e-5) * g_ref[...] + b_ref[...]
    hn_ref[...] = hn
    wa = wa_ref[...]
    wb = wb_ref[...]
    a_ref[...] = _pack_pair(_dotf(hn, wa[:, :64]), _dotf(hn, wa[:, 64:]))
    bb_ref[...] = _pack_pair(_dotf(hn, wb[:, :64]), _dotf(hn, wb[:, 64:]))


def _tc_pre(h, g1, b1, W1a, W1b):
    return pl.pallas_call(
        _pre_body,
        grid=(N // NB,),
        in_specs=[
            pl.BlockSpec((NB, D), lambda i: (i, 0)),
            pl.BlockSpec((1, D), lambda i: (0, 0)),
            pl.BlockSpec((1, D), lambda i: (0, 0)),
            pl.BlockSpec((D, D), lambda i: (0, 0)),
            pl.BlockSpec((D, D), lambda i: (0, 0)),
        ],
        out_specs=[
            pl.BlockSpec((NB, D), lambda i: (i, 0)),
            pl.BlockSpec((NB, D // 2), lambda i: (i, 0)),
            pl.BlockSpec((NB, D // 2), lambda i: (i, 0)),
        ],
        out_shape=[
            jax.ShapeDtypeStruct((N, D), jnp.float32),
            jax.ShapeDtypeStruct((N, D // 2), jnp.float32),
            jax.ShapeDtypeStruct((N, D // 2), jnp.float32),
        ],
    )(h, g1.reshape(1, D), b1.reshape(1, D), W1a, W1b)


def _sc_body(a_hbm, b_hbm, c_hbm, src_hbm, dst_hbm, out_hbm,
             sidx0, sidx1, sidx2, sidx3, didx0, didx1, didx2, didx3,
             bufa0, bufa1, bufb0, bufb1, bufc0, bufc1, buft0, buft1, acc,
             ssi0, ssi1, ssi2, ssi3, sdi0, sdi1, sdi2, sdi3,
             sa0, sa1, sb0, sb1, sc0, sc1, sst0, sst1):
    sidx = (sidx0, sidx1, sidx2, sidx3)
    didx = (didx0, didx1, didx2, didx3)
    bufa = (bufa0, bufa1)
    bufb = (bufb0, bufb1)
    bufc = (bufc0, bufc1)
    buft = (buft0, buft1)
    sem_si = (ssi0, ssi1, ssi2, ssi3)
    sem_di = (sdi0, sdi1, sdi2, sdi3)
    sem_a = (sa0, sa1)
    sem_b = (sb0, sb1)
    sem_c = (sc0, sc1)
    sem_s = (sst0, sst1)

    cid = lax.axis_index("c")
    sid = lax.axis_index("s")
    zeros16 = jnp.zeros((16,), jnp.float32)

    def zrow(r, carry):
        for j in range(DP // 16):
            buft0[r, pl.ds(j * 16, 16)] = zeros16
        return carry

    lax.fori_loop(0, BK, zrow, 0)

    row0 = sid * RPT
    for q in range(RPT // BK):
        pltpu.sync_copy(buft0, acc.at[pl.ds(row0 + q * BK, BK)])

    onehot = jnp.where(lax.iota(jnp.int32, 16) == 0,
                       jnp.float32(1.0), jnp.float32(0.0))

    def trow(r, carry):
        buft0[r, pl.ds(D, 16)] = onehot
        buft1[r, pl.ds(D, 16)] = onehot
        return carry

    lax.fori_loop(0, BK, trow, 0)

    plsc.subcore_barrier()

    base = (cid * NS + sid) * EPT

    def issue_idx(jb, q):
        off = base + jb * BK
        pltpu.async_copy(src_hbm.at[pl.ds(off, BK)], sidx[q], sem_si[q])
        pltpu.async_copy(dst_hbm.at[pl.ds(off, BK)], didx[q], sem_di[q])

    def wait_idx(q):
        pltpu.make_async_copy(src_hbm.at[pl.ds(0, BK)], sidx[q],
                              sem_si[q]).wait()
        pltpu.make_async_copy(dst_hbm.at[pl.ds(0, BK)], didx[q],
                              sem_di[q]).wait()

    def issue_gathers(jb, q, p):
        off = base + jb * BK
        pltpu.async_copy(a_hbm.at[sidx[q]], bufa[p], sem_a[p])
        pltpu.async_copy(b_hbm.at[didx[q]], bufb[p], sem_b[p])
        pltpu.async_copy(c_hbm.at[pl.ds(off, BK)], bufc[p], sem_c[p])

    def wait_gathers(p):
        pltpu.make_async_copy(a_hbm.at[sidx[0]], bufa[p], sem_a[p]).wait()
        pltpu.make_async_copy(b_hbm.at[didx[0]], bufb[p], sem_b[p]).wait()
        pltpu.make_async_copy(c_hbm.at[pl.ds(0, BK)], bufc[p], sem_c[p]).wait()

    def issue_scatter(q, p):
        pltpu.async_copy(buft[p], acc.at[didx[q]], sem_s[p], add=True)

    def wait_scatter(p):
        pltpu.make_async_copy(buft[p], acc.at[didx[0]], sem_s[p]).wait()

    def compute(p):
        @plsc.parallel_loop(0, BK, step=1, unroll=4)
        def erow(e):
            fmt = plsc.PackFormat.INTERLEAVED
            for g in range(D // 32):
                sl = pl.ds(16 * g, 16)
                a0, a1 = plsc.unpack(
                    plsc.bitcast(bufa[p][e, sl], jnp.bfloat16), format=fmt)
                b0, b1 = plsc.unpack(
                    plsc.bitcast(bufb[p][e, sl], jnp.bfloat16), format=fmt)
                c0, c1 = plsc.unpack(
                    plsc.bitcast(bufc[p][e, sl], jnp.bfloat16), format=fmt)
                buft[p][e, pl.ds(32 * g, 16)] = _silu_poly(a0 + b0 + c0)
                buft[p][e, pl.ds(32 * g + 16, 16)] = _silu_poly(a1 + b1 + c1)

    def step(jb, q, p, guard_jo=None, issue_next=True, issue_idx2=True):
        qn = (q + 1) % 4
        pn = 1 - p
        if issue_next:
            wait_idx(qn)
        wait_gathers(p)
        if issue_next:
            issue_gathers(jb + 1, qn, pn)
        if guard_jo is None:
            wait_scatter(p)
        else:
            @pl.when(guard_jo >= 1)
            def _():
                wait_scatter(p)
        compute(p)
        issue_scatter(q, p)
        if issue_idx2:
            issue_idx(jb + 2, (q + 2) % 4)

    issue_idx(0, 0)
    wait_idx(0)
    issue_gathers(0, 0, 0)
    issue_idx(1, 1)

    def quad(jo, carry):
        for q in range(4):
            step(4 * jo + q, q, q % 2, guard_jo=jo if q < 2 else None)
        return carry

    lax.fori_loop(0, NBLK // 4, quad, 0)

    step(NBLK - 2, 0, 0, issue_idx2=False)
    step(NBLK - 1, 1, 1, issue_next=False, issue_idx2=False)
    wait_scatter(0)
    wait_scatter(1)

    plsc.subcore_barrier()
    pltpu.sync_copy(acc.at[pl.ds(row0, RPT)],
                    out_hbm.at[cid, pl.ds(row0, RPT)])


def _sc_edge(A, B, C, src, dst):
    mesh = plsc.VectorSubcoreMesh(core_axis_name="c", subcore_axis_name="s",
                                  num_cores=NC, num_subcores=NS)
    f = functools.partial(
        pl.kernel,
        out_type=jax.ShapeDtypeStruct((NC, NA, DP), jnp.float32),
        mesh=mesh,
        compiler_params=pltpu.CompilerParams(use_tc_tiling_on_sc=False,
                                             needs_layout_passes=False),
        scratch_types=[
            pltpu.VMEM((BK,), jnp.int32),
            pltpu.VMEM((BK,), jnp.int32),
            pltpu.VMEM((BK,), jnp.int32),
            pltpu.VMEM((BK,), jnp.int32),
            pltpu.VMEM((BK,), jnp.int32),
            pltpu.VMEM((BK,), jnp.int32),
            pltpu.VMEM((BK,), jnp.int32),
            pltpu.VMEM((BK,), jnp.int32),
            pltpu.VMEM((BK, D // 2), jnp.float32),
            pltpu.VMEM((BK, D // 2), jnp.float32),
            pltpu.VMEM((BK, D // 2), jnp.float32),
            pltpu.VMEM((BK, D // 2), jnp.float32),
            pltpu.VMEM((BK, D // 2), jnp.float32),
            pltpu.VMEM((BK, D // 2), jnp.float32),
            pltpu.VMEM((BK, DP), jnp.float32),
            pltpu.VMEM((BK, DP), jnp.float32),
            pltpu.VMEM_SHARED((NA, DP), jnp.float32),
        ] + [pltpu.SemaphoreType.DMA] * 16,
    )(_sc_body)
    return f(A, B, C, src, dst)


def _post_body(h_ref, hn_ref, p_ref, w2_ref, b2_ref, g2_ref, bb2_ref,
               nw1_ref, nb1_ref, nw2_ref, nb2_ref, o_ref):
    p = p_ref[0] + p_ref[1]
    t = p[:, :D]
    deg = p[:, D:D + 1]
    scale = 1.0 / jnp.maximum(deg, 1.0)
    agg = _dotf(t * scale, w2_ref[...]) + (deg * scale) * b2_ref[...]
    m = jnp.mean(agg, axis=-1, keepdims=True)
    v = jnp.mean(jnp.square(agg - m), axis=-1, keepdims=True)
    agg = (agg - m) / jnp.sqrt(v + 1e-5) * g2_ref[...] + bb2_ref[...]
    nw1 = nw1_ref[...]
    pre = _dotf(hn_ref[...], nw1[:D]) + _dotf(agg, nw1[D:]) + nb1_ref[...]
    act = pre * jax.nn.sigmoid(pre)
    o_ref[...] = h_ref[...] + _dotf(act, nw2_ref[...]) + nb2_ref[...]


def _tc_post(h, hn, P, eW2i, eb2i, g2, b2, nW1i, nb1i, nW2i, nb2i):
    return pl.pallas_call(
        _post_body,
        grid=(N // NB,),
        in_specs=[
            pl.BlockSpec((NB, D), lambda i: (i, 0)),
            pl.BlockSpec((NB, D), lambda i: (i, 0)),
            pl.BlockSpec((NC, NB, DP), lambda i: (0, i, 0)),
            pl.BlockSpec((D, D), lambda i: (0, 0)),
            pl.BlockSpec((1, D), lambda i: (0, 0)),
            pl.BlockSpec((1, D), lambda i: (0, 0)),
            pl.BlockSpec((1, D), lambda i: (0, 0)),
            pl.BlockSpec((2 * D, 2 * D), lambda i: (0, 0)),
            pl.BlockSpec((1, 2 * D), lambda i: (0, 0)),
            pl.BlockSpec((2 * D, D), lambda i: (0, 0)),
            pl.BlockSpec((1, D), lambda i: (0, 0)),
        ],
        out_specs=pl.BlockSpec((NB, D), lambda i: (i, 0)),
        out_shape=jax.ShapeDtypeStruct((N, D), jnp.float32),
    )(h, hn, P, eW2i, eb2i.reshape(1, D), g2.reshape(1, D), b2.reshape(1, D),
      nW1i, nb1i.reshape(1, 2 * D), nW2i, nb2i.reshape(1, D))


def kernel(node_state, edge_index, edge_attr, ln1_g, ln1_b, ln2_g, ln2_b,
           eW1, eb1, eW2, eb2, nW1, nb1, nW2, nb2):
    src = edge_index[0]
    dst = edge_index[1]
    W1c = eW1[:, 2 * D:, :]
    C01 = _tc_edge_bias(edge_attr, W1c, eb1)
    h = node_state
    for i in range(2):
        hn, A, B = _tc_pre(h, ln1_g[i], ln1_b[i], eW1[i, :D], eW1[i, D:2 * D])
        P = _sc_edge(A, B, C01[i], src, dst)
        h = _tc_post(h, hn, P, eW2[i][_PERM], eb2[i], ln2_g[i], ln2_b[i],
                     nW1[i], nb1[i], nW2[i], nb2[i])
    return h


# speedup vs baseline: 1.0978x; 1.0299x over previous
import functools

import jax
import jax.numpy as jnp
import numpy as np
from jax import lax
from jax.experimental import pallas as pl
from jax.experimental.pallas import tpu as pltpu
from jax.experimental.pallas import tpu_sc as plsc

N = 10000
E = 320000
D = 128
ED = 16
NC, NS = 2, 16
NW = NC * NS
EPT = E // NW
BK = 40
NBLK = EPT // BK
DP = D + 16
NA = 10240
RPT = NA // NS

_SILU_C = (0.0005758678889833391, 0.2481342852115631, -0.019295798614621162,
           0.001511988928541541, -8.809041901258752e-05, 3.3523247111588717e-06,
           -7.264971912945839e-08, 6.729672374916618e-10)


_PERM = np.concatenate([
    np.concatenate([np.arange(16 * g, 16 * g + 16),
                    np.arange(64 + 16 * g, 64 + 16 * g + 16)])
    for g in range(D // 32)
]).astype(np.int32)


def _silu_poly(x):
    c0, c1, c2, c3, c4, c5, c6, c7 = _SILU_C
    u = jnp.minimum(x * x, 25.0)
    u2 = u * u
    u4 = u2 * u2
    lo = (c0 + c1 * u) + u2 * (c2 + c3 * u)
    hi = (c4 + c5 * u) + u2 * (c6 + c7 * u)
    p = lo + u4 * hi
    return p + 0.5 * x + 0.5 * jnp.maximum(jnp.abs(x) - 5.0, 0.0)
NB = 2000
EB = 8000

_HI = lax.Precision.HIGHEST


def _dot(a, b):
    return jnp.dot(a, b, preferred_element_type=jnp.float32, precision=_HI)


def _dotf(a, b):
    return jnp.dot(a, b, preferred_element_type=jnp.float32)


def _pack_pair(lo_f32, hi_f32):
    lo = lax.bitcast_convert_type(lo_f32.astype(jnp.bfloat16), jnp.uint16)
    hi = lax.bitcast_convert_type(hi_f32.astype(jnp.bfloat16), jnp.uint16)
    word = lo.astype(jnp.uint32) | (hi.astype(jnp.uint32) << 16)
    return lax.bitcast_convert_type(word, jnp.float32)


def _c_body(ea_ref, w_ref, b_ref, c0_ref, c1_ref):
    ea = ea_ref[...]
    for i, cref in ((0, c0_ref), (1, c1_ref)):
        lo = _dotf(ea, w_ref[i, :, :64]) + b_ref[i, :64]
        hi = _dotf(ea, w_ref[i, :, 64:]) + b_ref[i, 64:]
        cref[...] = _pack_pair(lo, hi)


def _tc_edge_bias(edge_attr, W1c, eb1):
    return pl.pallas_call(
        _c_body,
        grid=(E // EB,),
        in_specs=[
            pl.BlockSpec((EB, ED), lambda i: (i, 0)),
            pl.BlockSpec((2, ED, D), lambda i: (0, 0, 0)),
            pl.BlockSpec((2, D), lambda i: (0, 0)),
        ],
        out_specs=[
            pl.BlockSpec((EB, D // 2), lambda i: (i, 0)),
            pl.BlockSpec((EB, D // 2), lambda i: (i, 0)),
        ],
        out_shape=[
            jax.ShapeDtypeStruct((E, D // 2), jnp.float32),
            jax.ShapeDtypeStruct((E, D // 2), jnp.float32),
        ],
    )(edge_attr, W1c, eb1)


def _pre_body(h_ref, g_ref, b_ref, wa_ref, wb_ref, hn_ref, a_ref, bb_ref):
    x = h_ref[...]
    m = jnp.mean(x, axis=-1, keepdims=True)
    v = jnp.mean(jnp.square(x - m), axis=-1, keepdims=True)
    hn = (x - m) / jnp.sqrt(v + 1e-5) * g_ref[...] + b_ref[...]
    hn_ref[...] = hn
    wa = wa_ref[...]
    wb = wb_ref[...]
    a_ref[...] = _pack_pair(_dotf(hn, wa[:, :64]), _dotf(hn, wa[:, 64:]))
    bb_ref[...] = _pack_pair(_dotf(hn, wb[:, :64]), _dotf(hn, wb[:, 64:]))


def _tc_pre(h, g1, b1, W1a, W1b):
    return pl.pallas_call(
        _pre_body,
        grid=(N // NB,),
        in_specs=[
            pl.BlockSpec((NB, D), lambda i: (i, 0)),
            pl.BlockSpec((1, D), lambda i: (0, 0)),
            pl.BlockSpec((1, D), lambda i: (0, 0)),
            pl.BlockSpec((D, D), lambda i: (0, 0)),
            pl.BlockSpec((D, D), lambda i: (0, 0)),
        ],
        out_specs=[
            pl.BlockSpec((NB, D), lambda i: (i, 0)),
            pl.BlockSpec((NB, D // 2), lambda i: (i, 0)),
            pl.BlockSpec((NB, D // 2), lambda i: (i, 0)),
        ],
        out_shape=[
            jax.ShapeDtypeStruct((N, D), jnp.float32),
            jax.ShapeDtypeStruct((N, D // 2), jnp.float32),
            jax.ShapeDtypeStruct((N, D // 2), jnp.float32),
        ],
    )(h, g1.reshape(1, D), b1.reshape(1, D), W1a, W1b)


def _sc_body(a_hbm, b_hbm, c_hbm, src_hbm, dst_hbm, out_hbm,
             sidx0, sidx1, sidx2, sidx3, didx0, didx1, didx2, didx3,
             bufa0, bufa1, bufb0, bufb1, bufc0, bufc1, buft0, buft1, acc,
             ssi0, ssi1, ssi2, ssi3, sdi0, sdi1, sdi2, sdi3,
             sa0, sa1, sb0, sb1, sc0, sc1, sst0, sst1):
    sidx = (sidx0, sidx1, sidx2, sidx3)
    didx = (didx0, didx1, didx2, didx3)
    bufa = (bufa0, bufa1)
    bufb = (bufb0, bufb1)
    bufc = (bufc0, bufc1)
    buft = (buft0, buft1)
    sem_si = (ssi0, ssi1, ssi2, ssi3)
    sem_di = (sdi0, sdi1, sdi2, sdi3)
    sem_a = (sa0, sa1)
    sem_b = (sb0, sb1)
    sem_c = (sc0, sc1)
    sem_s = (sst0, sst1)

    cid = lax.axis_index("c")
    sid = lax.axis_index("s")
    zeros16 = jnp.zeros((16,), jnp.float32)

    def zrow(r, carry):
        for j in range(DP // 16):
            buft0[r, pl.ds(j * 16, 16)] = zeros16
        return carry

    lax.fori_loop(0, BK, zrow, 0)

    row0 = sid * RPT
    for q in range(RPT // BK):
        pltpu.sync_copy(buft0, acc.at[pl.ds(row0 + q * BK, BK)])

    onehot = jnp.where(lax.iota(jnp.int32, 16) == 0,
                       jnp.float32(1.0), jnp.float32(0.0))

    def trow(r, carry):
        buft0[r, pl.ds(D, 16)] = onehot
        buft1[r, pl.ds(D, 16)] = onehot
        return carry

    lax.fori_loop(0, BK, trow, 0)

    plsc.subcore_barrier()

    base = (cid * NS + sid) * EPT

    def issue_idx(jb, q):
        off = base + jb * BK
        pltpu.async_copy(src_hbm.at[pl.ds(off, BK)], sidx[q], sem_si[q])
        pltpu.async_copy(dst_hbm.at[pl.ds(off, BK)], didx[q], sem_di[q])

    def wait_idx(q):
        pltpu.make_async_copy(src_hbm.at[pl.ds(0, BK)], sidx[q],
                              sem_si[q]).wait()
        pltpu.make_async_copy(dst_hbm.at[pl.ds(0, BK)], didx[q],
                              sem_di[q]).wait()

    def issue_gathers(jb, q, p):
        off = base + jb * BK
        pltpu.async_copy(a_hbm.at[sidx[q]], bufa[p], sem_a[p])
        pltpu.async_copy(b_hbm.at[didx[q]], bufb[p], sem_b[p])
        pltpu.async_copy(c_hbm.at[pl.ds(off, BK)], bufc[p], sem_c[p])

    def wait_gathers(p):
        pltpu.make_async_copy(a_hbm.at[sidx[0]], bufa[p], sem_a[p]).wait()
        pltpu.make_async_copy(b_hbm.at[didx[0]], bufb[p], sem_b[p]).wait()
        pltpu.make_async_copy(c_hbm.at[pl.ds(0, BK)], bufc[p], sem_c[p]).wait()

    def issue_scatter(q, p):
        pltpu.async_copy(buft[p], acc.at[didx[q]], sem_s[p], add=True)

    def wait_scatter(p):
        pltpu.make_async_copy(buft[p], acc.at[didx[0]], sem_s[p]).wait()

    def compute(p):
        @plsc.parallel_loop(0, BK, step=1, unroll=4)
        def erow(e):
            fmt = plsc.PackFormat.INTERLEAVED
            for g in range(D // 32):
                sl = pl.ds(16 * g, 16)
                a32 = plsc.bitcast(bufa[p][e, sl], jnp.bfloat16)
                b32 = plsc.bitcast(bufb[p][e, sl], jnp.bfloat16)
                c32 = plsc.bitcast(bufc[p][e, sl], jnp.bfloat16)
                x0, x1 = plsc.unpack((a32 + b32) + c32, format=fmt)
                buft[p][e, pl.ds(32 * g, 16)] = _silu_poly(x0)
                buft[p][e, pl.ds(32 * g + 16, 16)] = _silu_poly(x1)

    def step(jb, q, p, guard_jo=None, issue_next=True, issue_idx2=True):
        qn = (q + 1) % 4
        pn = 1 - p
        if issue_next:
            wait_idx(qn)
        wait_gathers(p)
        if issue_next:
            issue_gathers(jb + 1, qn, pn)
        if guard_jo is None:
            wait_scatter(p)
        else:
            @pl.when(guard_jo >= 1)
            def _():
                wait_scatter(p)
        compute(p)
        issue_scatter(q, p)
        if issue_idx2:
            issue_idx(jb + 2, (q + 2) % 4)

    issue_idx(0, 0)
    wait_idx(0)
    issue_gathers(0, 0, 0)
    issue_idx(1, 1)

    def quad(jo, carry):
        for q in range(4):
            step(4 * jo + q, q, q % 2, guard_jo=jo if q < 2 else None)
        return carry

    lax.fori_loop(0, NBLK // 4, quad, 0)

    step(NBLK - 2, 0, 0, issue_idx2=False)
    step(NBLK - 1, 1, 1, issue_next=False, issue_idx2=False)
    wait_scatter(0)
    wait_scatter(1)

    plsc.subcore_barrier()
    pltpu.sync_copy(acc.at[pl.ds(row0, RPT)],
                    out_hbm.at[cid, pl.ds(row0, RPT)])


def _sc_edge(A, B, C, src, dst):
    mesh = plsc.VectorSubcoreMesh(core_axis_name="c", subcore_axis_name="s",
                                  num_cores=NC, num_subcores=NS)
    f = functools.partial(
        pl.kernel,
        out_type=jax.ShapeDtypeStruct((NC, NA, DP), jnp.float32),
        mesh=mesh,
        compiler_params=pltpu.CompilerParams(use_tc_tiling_on_sc=False,
                                             needs_layout_passes=False),
        scratch_types=[
            pltpu.VMEM((BK,), jnp.int32),
            pltpu.VMEM((BK,), jnp.int32),
            pltpu.VMEM((BK,), jnp.int32),
            pltpu.VMEM((BK,), jnp.int32),
            pltpu.VMEM((BK,), jnp.int32),
            pltpu.VMEM((BK,), jnp.int32),
            pltpu.VMEM((BK,), jnp.int32),
            pltpu.VMEM((BK,), jnp.int32),
            pltpu.VMEM((BK, D // 2), jnp.float32),
            pltpu.VMEM((BK, D // 2), jnp.float32),
            pltpu.VMEM((BK, D // 2), jnp.float32),
            pltpu.VMEM((BK, D // 2), jnp.float32),
            pltpu.VMEM((BK, D // 2), jnp.float32),
            pltpu.VMEM((BK, D // 2), jnp.float32),
            pltpu.VMEM((BK, DP), jnp.float32),
            pltpu.VMEM((BK, DP), jnp.float32),
            pltpu.VMEM_SHARED((NA, DP), jnp.float32),
        ] + [pltpu.SemaphoreType.DMA] * 16,
    )(_sc_body)
    return f(A, B, C, src, dst)


def _post_body(h_ref, hn_ref, p_ref, w2_ref, b2_ref, g2_ref, bb2_ref,
               nw1_ref, nb1_ref, nw2_ref, nb2_ref, o_ref):
    p = p_ref[0] + p_ref[1]
    t = p[:, :D]
    deg = p[:, D:D + 1]
    scale = 1.0 / jnp.maximum(deg, 1.0)
    agg = _dotf(t * scale, w2_ref[...]) + (deg * scale) * b2_ref[...]
    m = jnp.mean(agg, axis=-1, keepdims=True)
    v = jnp.mean(jnp.square(agg - m), axis=-1, keepdims=True)
    agg = (agg - m) / jnp.sqrt(v + 1e-5) * g2_ref[...] + bb2_ref[...]
    nw1 = nw1_ref[...]
    pre = _dotf(hn_ref[...], nw1[:D]) + _dotf(agg, nw1[D:]) + nb1_ref[...]
    act = pre * jax.nn.sigmoid(pre)
    o_ref[...] = h_ref[...] + _dotf(act, nw2_ref[...]) + nb2_ref[...]


def _tc_post(h, hn, P, eW2i, eb2i, g2, b2, nW1i, nb1i, nW2i, nb2i):
    return pl.pallas_call(
        _post_body,
        grid=(N // NB,),
        in_specs=[
            pl.BlockSpec((NB, D), lambda i: (i, 0)),
            pl.BlockSpec((NB, D), lambda i: (i, 0)),
            pl.BlockSpec((NC, NB, DP), lambda i: (0, i, 0)),
            pl.BlockSpec((D, D), lambda i: (0, 0)),
            pl.BlockSpec((1, D), lambda i: (0, 0)),
            pl.BlockSpec((1, D), lambda i: (0, 0)),
            pl.BlockSpec((1, D), lambda i: (0, 0)),
            pl.BlockSpec((2 * D, 2 * D), lambda i: (0, 0)),
            pl.BlockSpec((1, 2 * D), lambda i: (0, 0)),
            pl.BlockSpec((2 * D, D), lambda i: (0, 0)),
            pl.BlockSpec((1, D), lambda i: (0, 0)),
        ],
        out_specs=pl.BlockSpec((NB, D), lambda i: (i, 0)),
        out_shape=jax.ShapeDtypeStruct((N, D), jnp.float32),
    )(h, hn, P, eW2i, eb2i.reshape(1, D), g2.reshape(1, D), b2.reshape(1, D),
      nW1i, nb1i.reshape(1, 2 * D), nW2i, nb2i.reshape(1, D))


def kernel(node_state, edge_index, edge_attr, ln1_g, ln1_b, ln2_g, ln2_b,
           eW1, eb1, eW2, eb2, nW1, nb1, nW2, nb2):
    src = edge_index[0]
    dst = edge_index[1]
    W1c = eW1[:, 2 * D:, :]
    C01 = _tc_edge_bias(edge_attr, W1c, eb1)
    h = node_state
    for i in range(2):
        hn, A, B = _tc_pre(h, ln1_g[i], ln1_b[i], eW1[i, :D], eW1[i, D:2 * D])
        P = _sc_edge(A, B, C01[i], src, dst)
        h = _tc_post(h, hn, P, eW2[i][_PERM], eb2[i], ln2_g[i], ln2_b[i],
                     nW1[i], nb1[i], nW2[i], nb2[i])
    return h


# speedup vs baseline: 1.1142x; 1.0150x over previous
import functools

import jax
import jax.numpy as jnp
import numpy as np
from jax import lax
from jax.experimental import pallas as pl
from jax.experimental.pallas import tpu as pltpu
from jax.experimental.pallas import tpu_sc as plsc

N = 10000
E = 320000
D = 128
ED = 16
NC, NS = 2, 16
NW = NC * NS
EPT = E // NW
BK = 40
NBLK = EPT // BK
DP = D + 16
NA = 10240
RPT = NA // NS

_SILU_C = (0.0005758678889833391, 0.2481342852115631, -0.019295798614621162,
           0.001511988928541541, -8.809041901258752e-05, 3.3523247111588717e-06,
           -7.264971912945839e-08, 6.729672374916618e-10)


_PERM = np.concatenate([
    np.concatenate([np.arange(16 * g, 16 * g + 16),
                    np.arange(64 + 16 * g, 64 + 16 * g + 16)])
    for g in range(D // 32)
]).astype(np.int32)


def _silu_poly(x):
    c0, c1, c2, c3, c4, c5, c6, c7 = _SILU_C
    u = jnp.minimum(x * x, 25.0)
    u2 = u * u
    u4 = u2 * u2
    lo = (c0 + c1 * u) + u2 * (c2 + c3 * u)
    hi = (c4 + c5 * u) + u2 * (c6 + c7 * u)
    p = lo + u4 * hi
    return p + 0.5 * x + 0.5 * jnp.maximum(jnp.abs(x) - 5.0, 0.0)
NB = 2000
EB = 8000

_HI = lax.Precision.HIGHEST


def _dot(a, b):
    return jnp.dot(a, b, preferred_element_type=jnp.float32, precision=_HI)


def _dotf(a, b):
    return jnp.dot(a, b, preferred_element_type=jnp.float32)


def _pack_pair(lo_f32, hi_f32):
    lo = lax.bitcast_convert_type(lo_f32.astype(jnp.bfloat16), jnp.uint16)
    hi = lax.bitcast_convert_type(hi_f32.astype(jnp.bfloat16), jnp.uint16)
    word = lo.astype(jnp.uint32) | (hi.astype(jnp.uint32) << 16)
    return lax.bitcast_convert_type(word, jnp.float32)


def _c_body(ea_ref, w_ref, b_ref, c_ref):
    ea = ea_ref[...]
    lo = _dotf(ea, w_ref[:, :64]) + b_ref[0, :64]
    hi = _dotf(ea, w_ref[:, 64:]) + b_ref[0, 64:]
    c_ref[...] = _pack_pair(lo, hi)


def _tc_edge_bias(edge_attr, W1ci, eb1i):
    return pl.pallas_call(
        _c_body,
        grid=(E // EB,),
        in_specs=[
            pl.BlockSpec((EB, ED), lambda i: (i, 0)),
            pl.BlockSpec((ED, D), lambda i: (0, 0)),
            pl.BlockSpec((1, D), lambda i: (0, 0)),
        ],
        out_specs=pl.BlockSpec((EB, D // 2), lambda i: (i, 0)),
        out_shape=jax.ShapeDtypeStruct((E, D // 2), jnp.float32),
    )(edge_attr, W1ci, eb1i.reshape(1, D))


def _pre_body(h_ref, g_ref, b_ref, wa_ref, wb_ref, hn_ref, a_ref, bb_ref):
    x = h_ref[...]
    m = jnp.mean(x, axis=-1, keepdims=True)
    v = jnp.mean(jnp.square(x - m), axis=-1, keepdims=True)
    hn = (x - m) / jnp.sqrt(v + 1e-5) * g_ref[...] + b_ref[...]
    hn_ref[...] = hn
    wa = wa_ref[...]
    wb = wb_ref[...]
    a_ref[...] = _pack_pair(_dotf(hn, wa[:, :64]), _dotf(hn, wa[:, 64:]))
    bb_ref[...] = _pack_pair(_dotf(hn, wb[:, :64]), _dotf(hn, wb[:, 64:]))


def _tc_pre(h, g1, b1, W1a, W1b):
    return pl.pallas_call(
        _pre_body,
        grid=(N // NB,),
        in_specs=[
            pl.BlockSpec((NB, D), lambda i: (i, 0)),
            pl.BlockSpec((1, D), lambda i: (0, 0)),
            pl.BlockSpec((1, D), lambda i: (0, 0)),
            pl.BlockSpec((D, D), lambda i: (0, 0)),
            pl.BlockSpec((D, D), lambda i: (0, 0)),
        ],
        out_specs=[
            pl.BlockSpec((NB, D), lambda i: (i, 0)),
            pl.BlockSpec((NB, D // 2), lambda i: (i, 0)),
            pl.BlockSpec((NB, D // 2), lambda i: (i, 0)),
        ],
        out_shape=[
            jax.ShapeDtypeStruct((N, D), jnp.float32),
            jax.ShapeDtypeStruct((N, D // 2), jnp.float32),
            jax.ShapeDtypeStruct((N, D // 2), jnp.float32),
        ],
    )(h, g1.reshape(1, D), b1.reshape(1, D), W1a, W1b)


def _sc_body(a_hbm, b_hbm, c_hbm, src_hbm, dst_hbm, out_hbm,
             sidx0, sidx1, sidx2, sidx3, didx0, didx1, didx2, didx3,
             bufa0, bufa1, bufb0, bufb1, bufc0, bufc1, buft0, buft1, acc,
             ssi0, ssi1, ssi2, ssi3, sdi0, sdi1, sdi2, sdi3,
             sa0, sa1, sb0, sb1, sc0, sc1, sst0, sst1):
    sidx = (sidx0, sidx1, sidx2, sidx3)
    didx = (didx0, didx1, didx2, didx3)
    bufa = (bufa0, bufa1)
    bufb = (bufb0, bufb1)
    bufc = (bufc0, bufc1)
    buft = (buft0, buft1)
    sem_si = (ssi0, ssi1, ssi2, ssi3)
    sem_di = (sdi0, sdi1, sdi2, sdi3)
    sem_a = (sa0, sa1)
    sem_b = (sb0, sb1)
    sem_c = (sc0, sc1)
    sem_s = (sst0, sst1)

    cid = lax.axis_index("c")
    sid = lax.axis_index("s")
    zeros16 = jnp.zeros((16,), jnp.float32)

    def zrow(r, carry):
        for j in range(DP // 16):
            buft0[r, pl.ds(j * 16, 16)] = zeros16
        return carry

    lax.fori_loop(0, BK, zrow, 0)

    row0 = sid * RPT
    for q in range(RPT // BK):
        pltpu.sync_copy(buft0, acc.at[pl.ds(row0 + q * BK, BK)])

    onehot = jnp.where(lax.iota(jnp.int32, 16) == 0,
                       jnp.float32(1.0), jnp.float32(0.0))

    def trow(r, carry):
        buft0[r, pl.ds(D, 16)] = onehot
        buft1[r, pl.ds(D, 16)] = onehot
        return carry

    lax.fori_loop(0, BK, trow, 0)

    plsc.subcore_barrier()

    base = (cid * NS + sid) * EPT

    def issue_idx(jb, q):
        off = base + jb * BK
        pltpu.async_copy(src_hbm.at[pl.ds(off, BK)], sidx[q], sem_si[q])
        pltpu.async_copy(dst_hbm.at[pl.ds(off, BK)], didx[q], sem_di[q])

    def wait_idx(q):
        pltpu.make_async_copy(src_hbm.at[pl.ds(0, BK)], sidx[q],
                              sem_si[q]).wait()
        pltpu.make_async_copy(dst_hbm.at[pl.ds(0, BK)], didx[q],
                              sem_di[q]).wait()

    def issue_gathers(jb, q, p):
        off = base + jb * BK
        pltpu.async_copy(a_hbm.at[sidx[q]], bufa[p], sem_a[p])
        pltpu.async_copy(b_hbm.at[didx[q]], bufb[p], sem_b[p])
        pltpu.async_copy(c_hbm.at[pl.ds(off, BK)], bufc[p], sem_c[p])

    def wait_gathers(p):
        pltpu.make_async_copy(a_hbm.at[sidx[0]], bufa[p], sem_a[p]).wait()
        pltpu.make_async_copy(b_hbm.at[didx[0]], bufb[p], sem_b[p]).wait()
        pltpu.make_async_copy(c_hbm.at[pl.ds(0, BK)], bufc[p], sem_c[p]).wait()

    def issue_scatter(q, p):
        pltpu.async_copy(buft[p], acc.at[didx[q]], sem_s[p], add=True)

    def wait_scatter(p):
        pltpu.make_async_copy(buft[p], acc.at[didx[0]], sem_s[p]).wait()

    def compute(p):
        @plsc.parallel_loop(0, BK, step=1, unroll=4)
        def erow(e):
            fmt = plsc.PackFormat.INTERLEAVED
            for g in range(D // 32):
                sl = pl.ds(16 * g, 16)
                a32 = plsc.bitcast(bufa[p][e, sl], jnp.bfloat16)
                b32 = plsc.bitcast(bufb[p][e, sl], jnp.bfloat16)
                c32 = plsc.bitcast(bufc[p][e, sl], jnp.bfloat16)
                x0, x1 = plsc.unpack((a32 + b32) + c32, format=fmt)
                buft[p][e, pl.ds(32 * g, 16)] = _silu_poly(x0)
                buft[p][e, pl.ds(32 * g + 16, 16)] = _silu_poly(x1)

    def step(jb, q, p, guard_jo=None, issue_next=True, issue_idx2=True):
        qn = (q + 1) % 4
        pn = 1 - p
        if issue_next:
            wait_idx(qn)
        wait_gathers(p)
        if issue_next:
            issue_gathers(jb + 1, qn, pn)
        if guard_jo is None:
            wait_scatter(p)
        else:
            @pl.when(guard_jo >= 1)
            def _():
                wait_scatter(p)
        compute(p)
        issue_scatter(q, p)
        if issue_idx2:
            issue_idx(jb + 2, (q + 2) % 4)

    issue_idx(0, 0)
    wait_idx(0)
    issue_gathers(0, 0, 0)
    issue_idx(1, 1)

    def quad(jo, carry):
        for q in range(4):
            step(4 * jo + q, q, q % 2, guard_jo=jo if q < 2 else None)
        return carry

    lax.fori_loop(0, NBLK // 4, quad, 0)

    step(NBLK - 2, 0, 0, issue_idx2=False)
    step(NBLK - 1, 1, 1, issue_next=False, issue_idx2=False)
    wait_scatter(0)
    wait_scatter(1)

    plsc.subcore_barrier()
    pltpu.sync_copy(acc.at[pl.ds(row0, RPT)],
                    out_hbm.at[cid, pl.ds(row0, RPT)])


def _sc_edge(A, B, C, src, dst):
    mesh = plsc.VectorSubcoreMesh(core_axis_name="c", subcore_axis_name="s",
                                  num_cores=NC, num_subcores=NS)
    f = functools.partial(
        pl.kernel,
        out_type=jax.ShapeDtypeStruct((NC, NA, DP), jnp.float32),
        mesh=mesh,
        compiler_params=pltpu.CompilerParams(use_tc_tiling_on_sc=False,
                                             needs_layout_passes=False),
        scratch_types=[
            pltpu.VMEM((BK,), jnp.int32),
            pltpu.VMEM((BK,), jnp.int32),
            pltpu.VMEM((BK,), jnp.int32),
            pltpu.VMEM((BK,), jnp.int32),
            pltpu.VMEM((BK,), jnp.int32),
            pltpu.VMEM((BK,), jnp.int32),
            pltpu.VMEM((BK,), jnp.int32),
            pltpu.VMEM((BK,), jnp.int32),
            pltpu.VMEM((BK, D // 2), jnp.float32),
            pltpu.VMEM((BK, D // 2), jnp.float32),
            pltpu.VMEM((BK, D // 2), jnp.float32),
            pltpu.VMEM((BK, D // 2), jnp.float32),
            pltpu.VMEM((BK, D // 2), jnp.float32),
            pltpu.VMEM((BK, D // 2), jnp.float32),
            pltpu.VMEM((BK, DP), jnp.float32),
            pltpu.VMEM((BK, DP), jnp.float32),
            pltpu.VMEM_SHARED((NA, DP), jnp.float32),
        ] + [pltpu.SemaphoreType.DMA] * 16,
    )(_sc_body)
    return f(A, B, C, src, dst)


def _post_body(h_ref, hn_ref, p_ref, w2_ref, b2_ref, g2_ref, bb2_ref,
               nw1_ref, nb1_ref, nw2_ref, nb2_ref, o_ref):
    p = p_ref[0] + p_ref[1]
    t = p[:, :D]
    deg = p[:, D:D + 1]
    scale = 1.0 / jnp.maximum(deg, 1.0)
    agg = _dotf(t * scale, w2_ref[...]) + (deg * scale) * b2_ref[...]
    m = jnp.mean(agg, axis=-1, keepdims=True)
    v = jnp.mean(jnp.square(agg - m), axis=-1, keepdims=True)
    agg = (agg - m) / jnp.sqrt(v + 1e-5) * g2_ref[...] + bb2_ref[...]
    nw1 = nw1_ref[...]
    pre = _dotf(hn_ref[...], nw1[:D]) + _dotf(agg, nw1[D:]) + nb1_ref[...]
    act = pre * jax.nn.sigmoid(pre)
    o_ref[...] = h_ref[...] + _dotf(act, nw2_ref[...]) + nb2_ref[...]


def _tc_post(h, hn, P, eW2i, eb2i, g2, b2, nW1i, nb1i, nW2i, nb2i):
    return pl.pallas_call(
        _post_body,
        grid=(N // NB,),
        in_specs=[
            pl.BlockSpec((NB, D), lambda i: (i, 0)),
            pl.BlockSpec((NB, D), lambda i: (i, 0)),
            pl.BlockSpec((NC, NB, DP), lambda i: (0, i, 0)),
            pl.BlockSpec((D, D), lambda i: (0, 0)),
            pl.BlockSpec((1, D), lambda i: (0, 0)),
            pl.BlockSpec((1, D), lambda i: (0, 0)),
            pl.BlockSpec((1, D), lambda i: (0, 0)),
            pl.BlockSpec((2 * D, 2 * D), lambda i: (0, 0)),
            pl.BlockSpec((1, 2 * D), lambda i: (0, 0)),
            pl.BlockSpec((2 * D, D), lambda i: (0, 0)),
            pl.BlockSpec((1, D), lambda i: (0, 0)),
        ],
        out_specs=pl.BlockSpec((NB, D), lambda i: (i, 0)),
        out_shape=jax.ShapeDtypeStruct((N, D), jnp.float32),
    )(h, hn, P, eW2i, eb2i.reshape(1, D), g2.reshape(1, D), b2.reshape(1, D),
      nW1i, nb1i.reshape(1, 2 * D), nW2i, nb2i.reshape(1, D))


def kernel(node_state, edge_index, edge_attr, ln1_g, ln1_b, ln2_g, ln2_b,
           eW1, eb1, eW2, eb2, nW1, nb1, nW2, nb2):
    src = edge_index[0]
    dst = edge_index[1]
    h = node_state
    for i in range(2):
        Ci = _tc_edge_bias(edge_attr, eW1[i, 2 * D:, :], eb1[i])
        hn, A, B = _tc_pre(h, ln1_g[i], ln1_b[i], eW1[i, :D], eW1[i, D:2 * D])
        P = _sc_edge(A, B, Ci, src, dst)
        h = _tc_post(h, hn, P, eW2[i][_PERM], eb2[i], ln2_g[i], ln2_b[i],
                     nW1[i], nb1[i], nW2[i], nb2[i])
    return h
